# baseline (speedup 1.0000x reference)
import functools

import jax
import jax.numpy as jnp
from jax import lax
from jax.experimental import pallas as pl
from jax.experimental.pallas import tpu as pltpu
from jax.experimental.pallas import tpu_sc as plsc

VOCAB = 1_000_000
EMBED_D = 64
BATCH = 4096
SEQ = 200
NUM_CORES = 2
NUM_SUBCORES = 16
D_PER_CORE = EMBED_D // NUM_CORES
B_PER_TILE = BATCH // NUM_SUBCORES
BP = 64
NPASS = B_PER_TILE // BP

_mesh = plsc.VectorSubcoreMesh(core_axis_name="c", subcore_axis_name="s")


@functools.partial(
    pl.kernel,
    mesh=_mesh,
    out_type=jax.ShapeDtypeStruct((SEQ, EMBED_D * BATCH), jnp.float32),
    scratch_types=[
        pltpu.VMEM_SHARED((VOCAB,), jnp.float32),
        [pltpu.VMEM((SEQ, BP), jnp.int32) for _ in range(NPASS)],
        pltpu.VMEM((SEQ, BP), jnp.float32),
        pltpu.SemaphoreType.DMA,
        pltpu.SemaphoreType.DMA,
    ],
    compiler_params=pltpu.CompilerParams(
        use_tc_tiling_on_sc=False, needs_layout_passes=False
    ),
)
def _dgather(table_t, x_t, out_hbm, td, idx_t, val, gsem, wsem):
    cid = lax.axis_index("c")
    sid = lax.axis_index("s")
    b0 = sid * B_PER_TILE

    for p in range(NPASS):
        pltpu.sync_copy(x_t.at[:, pl.ds(b0 + p * BP, BP)], idx_t[p])

    def dbody(k, _):
        dg = cid * D_PER_CORE + k

        @pl.when(sid == 0)
        def _():
            pltpu.sync_copy(table_t.at[dg], td)

        plsc.subcore_barrier()

        for p in range(NPASS):
            out_slab = out_hbm.at[:, pl.ds(dg * BATCH + b0 + p * BP, BP)]

            @pl.when(jnp.logical_or(k > 0, p > 0))
            def _():
                pltpu.make_async_copy(val, out_slab, wsem).wait()

            def grow(s, _):
                pltpu.make_async_copy(
                    td.at[idx_t[p].at[s]], val.at[s], gsem
                ).start()
                return ()

            lax.fori_loop(0, SEQ, grow, ())
            pltpu.make_async_copy(out_slab, val, gsem).wait()
            pltpu.make_async_copy(val, out_slab, wsem).start()

        plsc.subcore_barrier()
        return ()

    lax.fori_loop(0, D_PER_CORE, dbody, ())
    last = out_hbm.at[
        :, pl.ds((cid * D_PER_CORE + D_PER_CORE - 1) * BATCH
                 + b0 + (NPASS - 1) * BP, BP)
    ]
    pltpu.make_async_copy(val, last, wsem).wait()


def kernel(x, embeddings):
    out_t = _dgather(embeddings.T, x.T)
    return out_t.reshape(SEQ, EMBED_D, BATCH).transpose(2, 0, 1)
